# baseline (speedup 1.0000x reference)
import jax
import jax.numpy as jnp
from jax.experimental import pallas as pl
from jax.experimental.pallas import tpu as pltpu

_ROWS = 131072
_COLS = 257
_SLAB = 4096
_N = _ROWS // _SLAB
_KN = 4
_NCHUNK = _ROWS // _KN


def _pipeline_kernel(in_hbm, out_hbm, in_buf, out_buf, in_sems, out_sems, n_sems):
    i = pl.program_id(0)
    slot = jax.lax.rem(i, 2)
    nslot = jax.lax.rem(i + 1, 2)

    def in_wide(s, slot_):
        return pltpu.make_async_copy(
            in_hbm.at[pl.ds(s * _SLAB, _SLAB), pl.ds(0, 256)],
            in_buf.at[slot_], in_sems.at[slot_])

    def out_wide(s, slot_):
        return pltpu.make_async_copy(
            out_buf.at[slot_],
            out_hbm.at[pl.ds(s * _SLAB, _SLAB), pl.ds(0, 256)],
            out_sems.at[slot_])

    def narrow(k):
        return pltpu.make_async_copy(
            in_hbm.at[pl.ds(k * _NCHUNK, _NCHUNK), pl.ds(256, 1)],
            out_hbm.at[pl.ds(k * _NCHUNK, _NCHUNK), pl.ds(256, 1)],
            n_sems.at[k])

    @pl.when(i == 0)
    def _():
        for k in range(_KN):
            narrow(k).start()
        in_wide(i, slot).start()

    @pl.when(i + 1 < _N)
    def _():
        in_wide(i + 1, nslot).start()

    in_wide(i, slot).wait()

    @pl.when(i >= 2)
    def _():
        out_wide(i - 2, slot).wait()

    out_buf[slot, :, 0:128] = in_buf[slot, :, 128:256]
    out_buf[slot, :, 128:256] = in_buf[slot, :, 0:128]

    out_wide(i, slot).start()

    @pl.when(i == _N - 1)
    def _():
        out_wide(i - 1, nslot).wait()
        out_wide(i, slot).wait()
        for k in range(_KN):
            narrow(k).wait()


def kernel(tensor, list_ind):
    del list_ind
    return pl.pallas_call(
        _pipeline_kernel,
        grid=(_N,),
        in_specs=[pl.BlockSpec(memory_space=pl.ANY)],
        out_specs=pl.BlockSpec(memory_space=pl.ANY),
        out_shape=jax.ShapeDtypeStruct((_ROWS, _COLS), tensor.dtype),
        scratch_shapes=[
            pltpu.VMEM((2, _SLAB, 256), jnp.float32),
            pltpu.VMEM((2, _SLAB, 256), jnp.float32),
            pltpu.SemaphoreType.DMA((2,)),
            pltpu.SemaphoreType.DMA((2,)),
            pltpu.SemaphoreType.DMA((_KN,)),
        ],
    )(tensor)


# speedup vs baseline: 5.4045x; 5.4045x over previous
import jax
import jax.numpy as jnp
from jax.experimental import pallas as pl
from jax.experimental.pallas import tpu as pltpu

_ROWS = 131072
_COLS = 257
_SLAB = 4096
_N = _ROWS // _SLAB
_K = 4
_CH = _SLAB // _K


def _pipeline_kernel(in_hbm, out_hbm, in_buf, out_buf, in_sems, out_sems):
    i = pl.program_id(0)
    slot = jax.lax.rem(i, 2)
    nslot = jax.lax.rem(i + 1, 2)

    def in_copy(slab_idx, slot_, k):
        return pltpu.make_async_copy(
            in_hbm.at[pl.ds(slab_idx * _SLAB + k * _CH, _CH), :],
            in_buf.at[slot_, pl.ds(k * _CH, _CH), :],
            in_sems.at[slot_, k])

    def out_copy(slab_idx, slot_, k):
        return pltpu.make_async_copy(
            out_buf.at[slot_, pl.ds(k * _CH, _CH), :],
            out_hbm.at[pl.ds(slab_idx * _SLAB + k * _CH, _CH), :],
            out_sems.at[slot_, k])

    @pl.when(i == 0)
    def _():
        for k in range(_K):
            in_copy(i, slot, k).start(priority=k % 2)

    @pl.when(i + 1 < _N)
    def _():
        for k in range(_K):
            in_copy(i + 1, nslot, k).start(priority=k % 2)

    for k in range(_K):
        in_copy(i, slot, k).wait()

    @pl.when(i >= 2)
    def _():
        for k in range(_K):
            out_copy(i - 2, slot, k).wait()

    out_buf[slot, :, 0:128] = in_buf[slot, :, 128:256]
    out_buf[slot, :, 128:256] = in_buf[slot, :, 0:128]
    out_buf[slot, :, 256:257] = in_buf[slot, :, 256:257]

    for k in range(_K):
        out_copy(i, slot, k).start(priority=k % 2)

    @pl.when(i == _N - 1)
    def _():
        for k in range(_K):
            out_copy(i - 1, nslot, k).wait()
        for k in range(_K):
            out_copy(i, slot, k).wait()


def kernel(tensor, list_ind):
    del list_ind
    return pl.pallas_call(
        _pipeline_kernel,
        grid=(_N,),
        in_specs=[pl.BlockSpec(memory_space=pl.ANY)],
        out_specs=pl.BlockSpec(memory_space=pl.ANY),
        out_shape=jax.ShapeDtypeStruct((_ROWS, _COLS), tensor.dtype),
        scratch_shapes=[
            pltpu.VMEM((2, _SLAB, _COLS), jnp.float32),
            pltpu.VMEM((2, _SLAB, _COLS), jnp.float32),
            pltpu.SemaphoreType.DMA((2, _K)),
            pltpu.SemaphoreType.DMA((2, _K)),
        ],
    )(tensor)
